# initial kernel scaffold (unmeasured)
import functools

import jax
import jax.numpy as jnp
from jax import lax
from jax.experimental import pallas as pl
from jax.experimental.pallas import tpu as pltpu

N_DEV = 4


def kernel(x, dy):
    k_per, m = x.shape
    _, n = dy.shape
    chunk = m // N_DEV

    def body(x_ref, dy_ref, out_ref, acc_ref, send_buf, recv_buf,
             send_sems, recv_sems):
        p = lax.axis_index("i")
        left = lax.rem(p + N_DEV - 1, N_DEV)
        right = lax.rem(p + 1, N_DEV)

        barrier_sem = pltpu.get_barrier_semaphore()
        for nbr in (left, right):
            pl.semaphore_signal(
                barrier_sem, inc=1,
                device_id=(nbr,), device_id_type=pl.DeviceIdType.MESH,
            )
        pl.semaphore_wait(barrier_sem, 2)

        xb = x_ref[:, :].astype(jnp.bfloat16)
        dyb = dy_ref[:, :].astype(jnp.bfloat16)
        acc_ref[:, :] = lax.dot_general(
            xb, dyb, (((0,), (0,)), ((), ())),
            preferred_element_type=jnp.float32,
        )

        c0 = lax.rem(p + N_DEV - 1, N_DEV)
        send_buf[0, :, :] = (
            acc_ref[pl.ds(c0 * chunk, chunk), :].astype(jnp.bfloat16)
        )

        for h in range(N_DEV - 1):
            rdma = pltpu.make_async_remote_copy(
                src_ref=send_buf.at[h],
                dst_ref=recv_buf.at[h],
                send_sem=send_sems.at[h],
                recv_sem=recv_sems.at[h],
                device_id=(right,),
                device_id_type=pl.DeviceIdType.MESH,
            )
            rdma.start()
            rdma.wait()

            c = lax.rem(p + 2 - h + N_DEV, N_DEV)
            part = acc_ref[pl.ds(c * chunk, chunk), :]
            total = recv_buf[h, :, :].astype(jnp.float32) + part
            if h < N_DEV - 2:
                send_buf[h + 1, :, :] = total.astype(jnp.bfloat16)
            else:
                out_ref[:, :] = total

        @functools.partial(
            pl.run_scoped, exit_sem=pltpu.SemaphoreType.REGULAR
        )
        def _(exit_sem):
            for nbr in (left, right):
                pl.semaphore_signal(
                    exit_sem, inc=1,
                    device_id=(nbr,), device_id_type=pl.DeviceIdType.MESH,
                )
            pl.semaphore_wait(exit_sem, 2)

    return pl.pallas_call(
        body,
        out_shape=jax.ShapeDtypeStruct((chunk, n), jnp.float32),
        in_specs=[
            pl.BlockSpec(memory_space=pltpu.VMEM),
            pl.BlockSpec(memory_space=pltpu.VMEM),
        ],
        out_specs=pl.BlockSpec(memory_space=pltpu.VMEM),
        scratch_shapes=[
            pltpu.VMEM((m, n), jnp.float32),
            pltpu.VMEM((N_DEV - 1, chunk, n), jnp.bfloat16),
            pltpu.VMEM((N_DEV - 1, chunk, n), jnp.bfloat16),
            pltpu.SemaphoreType.DMA((N_DEV - 1,)),
            pltpu.SemaphoreType.DMA((N_DEV - 1,)),
        ],
        compiler_params=pltpu.CompilerParams(collective_id=0),
    )(x, dy)


# baseline (device time: 100851 ns/iter reference)
import functools

import jax
import jax.numpy as jnp
from jax import lax
from jax.experimental import pallas as pl
from jax.experimental.pallas import tpu as pltpu

N_DEV = 4


def kernel(x, dy):
    k_per, m = x.shape
    _, n = dy.shape
    chunk = m // N_DEV

    def body(x_ref, dy_ref, out_ref, acc_ref, send_buf, recv_buf,
             send_sems, recv_sems):
        p = lax.axis_index("i")
        left = lax.rem(p + N_DEV - 1, N_DEV)
        right = lax.rem(p + 1, N_DEV)

        barrier_sem = pltpu.get_barrier_semaphore()
        for nbr in (left, right):
            pl.semaphore_signal(
                barrier_sem, inc=1,
                device_id=(nbr,), device_id_type=pl.DeviceIdType.MESH,
            )
        pl.semaphore_wait(barrier_sem, 2)

        xb = x_ref[:, :].astype(jnp.bfloat16)
        dyb = dy_ref[:, :].astype(jnp.bfloat16)
        acc_ref[:, :] = lax.dot_general(
            xb, dyb, (((0,), (0,)), ((), ())),
            preferred_element_type=jnp.float32,
        )

        c0 = lax.rem(p + N_DEV - 1, N_DEV)
        send_buf[0, :, :] = (
            acc_ref[pl.ds(c0 * chunk, chunk), :].astype(jnp.bfloat16)
        )

        for h in range(N_DEV - 1):
            rdma = pltpu.make_async_remote_copy(
                src_ref=send_buf.at[h],
                dst_ref=recv_buf.at[h],
                send_sem=send_sems.at[h],
                recv_sem=recv_sems.at[h],
                device_id=(right,),
                device_id_type=pl.DeviceIdType.MESH,
            )
            rdma.start()
            rdma.wait()

            c = lax.rem(p + 2 - h + N_DEV, N_DEV)
            part = acc_ref[pl.ds(c * chunk, chunk), :]
            total = recv_buf[h, :, :].astype(jnp.float32) + part
            if h < N_DEV - 2:
                send_buf[h + 1, :, :] = total.astype(jnp.bfloat16)
            else:
                out_ref[:, :] = total

        @functools.partial(
            pl.run_scoped, exit_sem=pltpu.SemaphoreType.REGULAR
        )
        def _(exit_sem):
            for nbr in (left, right):
                pl.semaphore_signal(
                    exit_sem, inc=1,
                    device_id=(nbr,), device_id_type=pl.DeviceIdType.MESH,
                )
            pl.semaphore_wait(exit_sem, 2)

    return pl.pallas_call(
        body,
        out_shape=jax.ShapeDtypeStruct((chunk, n), jnp.float32),
        in_specs=[
            pl.BlockSpec(memory_space=pltpu.VMEM),
            pl.BlockSpec(memory_space=pltpu.VMEM),
        ],
        out_specs=pl.BlockSpec(memory_space=pltpu.VMEM),
        scratch_shapes=[
            pltpu.VMEM((m, n), jnp.float32),
            pltpu.VMEM((N_DEV - 1, chunk, n), jnp.bfloat16),
            pltpu.VMEM((N_DEV - 1, chunk, n), jnp.bfloat16),
            pltpu.SemaphoreType.DMA((N_DEV - 1,)),
            pltpu.SemaphoreType.DMA((N_DEV - 1,)),
        ],
        compiler_params=pltpu.CompilerParams(
            collective_id=0,
            vmem_limit_bytes=100 * 1024 * 1024,
        ),
    )(x, dy)


# device time: 61143 ns/iter; 1.6494x vs baseline; 1.6494x over previous
import functools

import jax
import jax.numpy as jnp
from jax import lax
from jax.experimental import pallas as pl
from jax.experimental.pallas import tpu as pltpu

N_DEV = 4


def kernel(x, dy):
    k_per, m = x.shape
    _, n = dy.shape
    chunk = m // N_DEV
    half = n // 2

    def body(x_ref, dy_ref, out_ref, xbT_ref, dyb_ref,
             send_r, recv_r, send_l, recv_l,
             ssem_r, rsem_r, ssem_l, rsem_l):
        p = lax.axis_index("i")
        left = lax.rem(p + N_DEV - 1, N_DEV)
        right = lax.rem(p + 1, N_DEV)

        barrier_sem = pltpu.get_barrier_semaphore()
        for nbr in (left, right):
            pl.semaphore_signal(
                barrier_sem, inc=1,
                device_id=(nbr,), device_id_type=pl.DeviceIdType.MESH,
            )
        pl.semaphore_wait(barrier_sem, 2)

        xbT_ref[:, :] = x_ref[:, :].astype(jnp.bfloat16).T
        dyb_ref[:, :] = dy_ref[:, :].astype(jnp.bfloat16)

        def partial_half(c, col0):
            xa = xbT_ref[pl.ds(c * chunk, chunk), :]
            db = dyb_ref[:, col0:col0 + half]
            return lax.dot_general(
                xa, db, (((1,), (0,)), ((), ())),
                preferred_element_type=jnp.float32,
            )

        send_r[0, :, :] = (
            partial_half(lax.rem(p + 3, N_DEV), 0).astype(jnp.bfloat16))
        send_l[0, :, :] = (
            partial_half(lax.rem(p + 1, N_DEV), half).astype(jnp.bfloat16))

        for h in range(N_DEV - 1):
            rdma_r = pltpu.make_async_remote_copy(
                src_ref=send_r.at[h], dst_ref=recv_r.at[h],
                send_sem=ssem_r.at[h], recv_sem=rsem_r.at[h],
                device_id=(right,), device_id_type=pl.DeviceIdType.MESH,
            )
            rdma_l = pltpu.make_async_remote_copy(
                src_ref=send_l.at[h], dst_ref=recv_l.at[h],
                send_sem=ssem_l.at[h], recv_sem=rsem_l.at[h],
                device_id=(left,), device_id_type=pl.DeviceIdType.MESH,
            )
            rdma_r.start()
            rdma_l.start()

            c_r = lax.rem(p + 2 - h + N_DEV, N_DEV)
            c_l = lax.rem(p + 2 + h, N_DEV)
            part_r = partial_half(c_r, 0)
            part_l = partial_half(c_l, half)

            rdma_r.wait()
            rdma_l.wait()

            tot_r = recv_r[h, :, :].astype(jnp.float32) + part_r
            tot_l = recv_l[h, :, :].astype(jnp.float32) + part_l
            if h < N_DEV - 2:
                send_r[h + 1, :, :] = tot_r.astype(jnp.bfloat16)
                send_l[h + 1, :, :] = tot_l.astype(jnp.bfloat16)
            else:
                out_ref[:, :half] = tot_r
                out_ref[:, half:] = tot_l

        @functools.partial(
            pl.run_scoped, exit_sem=pltpu.SemaphoreType.REGULAR
        )
        def _(exit_sem):
            for nbr in (left, right):
                pl.semaphore_signal(
                    exit_sem, inc=1,
                    device_id=(nbr,), device_id_type=pl.DeviceIdType.MESH,
                )
            pl.semaphore_wait(exit_sem, 2)

    comm_shape = (N_DEV - 1, chunk, half)
    return pl.pallas_call(
        body,
        out_shape=jax.ShapeDtypeStruct((chunk, n), jnp.float32),
        in_specs=[
            pl.BlockSpec(memory_space=pltpu.VMEM),
            pl.BlockSpec(memory_space=pltpu.VMEM),
        ],
        out_specs=pl.BlockSpec(memory_space=pltpu.VMEM),
        scratch_shapes=[
            pltpu.VMEM((m, k_per), jnp.bfloat16),
            pltpu.VMEM((k_per, n), jnp.bfloat16),
            pltpu.VMEM(comm_shape, jnp.bfloat16),
            pltpu.VMEM(comm_shape, jnp.bfloat16),
            pltpu.VMEM(comm_shape, jnp.bfloat16),
            pltpu.VMEM(comm_shape, jnp.bfloat16),
            pltpu.SemaphoreType.DMA((N_DEV - 1,)),
            pltpu.SemaphoreType.DMA((N_DEV - 1,)),
            pltpu.SemaphoreType.DMA((N_DEV - 1,)),
            pltpu.SemaphoreType.DMA((N_DEV - 1,)),
        ],
        compiler_params=pltpu.CompilerParams(
            collective_id=0,
            vmem_limit_bytes=100 * 1024 * 1024,
        ),
    )(x, dy)


# device time: 54615 ns/iter; 1.8466x vs baseline; 1.1195x over previous
import jax
import jax.numpy as jnp
from jax import lax
from jax.experimental import pallas as pl
from jax.experimental.pallas import tpu as pltpu

N_DEV = 4
NB = 2


def kernel(x, dy):
    k_per, m = x.shape
    _, n = dy.shape
    chunk = m // N_DEV
    half = n // 2
    bw = half // NB

    def body(x_ref, dy_ref, out_ref, xbT_ref, dyb_ref,
             send_r, recv_r, send_l, recv_l,
             ssem_r, rsem_r, ssem_l, rsem_l):
        p = lax.axis_index("i")
        left = lax.rem(p + N_DEV - 1, N_DEV)
        right = lax.rem(p + 1, N_DEV)

        xbT_ref[:, :] = x_ref[:, :].astype(jnp.bfloat16).T

        barrier_sem = pltpu.get_barrier_semaphore()
        for nbr in (left, right):
            pl.semaphore_signal(
                barrier_sem, inc=1,
                device_id=(nbr,), device_id_type=pl.DeviceIdType.MESH,
            )
        pl.semaphore_wait(barrier_sem, 2)

        def part_block(c, col0, convert_dy=False):
            xa = xbT_ref[pl.ds(c * chunk, chunk), :]
            if convert_dy:
                db = dy_ref[:, col0:col0 + bw].astype(jnp.bfloat16)
                dyb_ref[:, col0:col0 + bw] = db
            else:
                db = dyb_ref[:, col0:col0 + bw]
            return lax.dot_general(
                xa, db, (((1,), (0,)), ((), ())),
                preferred_element_type=jnp.float32,
            )

        def make_rdma(h, b, direction):
            if direction == "r":
                return pltpu.make_async_remote_copy(
                    src_ref=send_r.at[h, b], dst_ref=recv_r.at[h, b],
                    send_sem=ssem_r.at[h, b], recv_sem=rsem_r.at[h, b],
                    device_id=(right,),
                    device_id_type=pl.DeviceIdType.MESH,
                )
            return pltpu.make_async_remote_copy(
                src_ref=send_l.at[h, b], dst_ref=recv_l.at[h, b],
                send_sem=ssem_l.at[h, b], recv_sem=rsem_l.at[h, b],
                device_id=(left,),
                device_id_type=pl.DeviceIdType.MESH,
            )

        c_send_r0 = lax.rem(p + 3, N_DEV)
        c_send_l0 = lax.rem(p + 1, N_DEV)

        for b in range(NB):
            send_r[0, b, :, :] = (
                part_block(c_send_r0, b * bw, convert_dy=True)
                .astype(jnp.bfloat16))
            make_rdma(0, b, "r").start()
            send_l[0, b, :, :] = (
                part_block(c_send_l0, half + b * bw, convert_dy=True)
                .astype(jnp.bfloat16))
            make_rdma(0, b, "l").start()

        for h in range(N_DEV - 1):
            c_r = lax.rem(p + 2 - h + N_DEV, N_DEV)
            c_l = lax.rem(p + 2 + h, N_DEV)
            for b in range(NB):
                part_r = part_block(c_r, b * bw)
                part_l = part_block(c_l, half + b * bw)

                rdma_r = make_rdma(h, b, "r")
                rdma_l = make_rdma(h, b, "l")
                rdma_r.wait()
                tot_r = recv_r[h, b, :, :].astype(jnp.float32) + part_r
                if h < N_DEV - 2:
                    send_r[h + 1, b, :, :] = tot_r.astype(jnp.bfloat16)
                    make_rdma(h + 1, b, "r").start()
                else:
                    out_ref[:, b * bw:(b + 1) * bw] = tot_r

                rdma_l.wait()
                tot_l = recv_l[h, b, :, :].astype(jnp.float32) + part_l
                if h < N_DEV - 2:
                    send_l[h + 1, b, :, :] = tot_l.astype(jnp.bfloat16)
                    make_rdma(h + 1, b, "l").start()
                else:
                    out_ref[:, half + b * bw:half + (b + 1) * bw] = tot_l

    comm_shape = (N_DEV - 1, NB, chunk, bw)
    sem_shape = (N_DEV - 1, NB)
    return pl.pallas_call(
        body,
        out_shape=jax.ShapeDtypeStruct((chunk, n), jnp.float32),
        in_specs=[
            pl.BlockSpec(memory_space=pltpu.VMEM),
            pl.BlockSpec(memory_space=pltpu.VMEM),
        ],
        out_specs=pl.BlockSpec(memory_space=pltpu.VMEM),
        scratch_shapes=[
            pltpu.VMEM((m, k_per), jnp.bfloat16),
            pltpu.VMEM((k_per, n), jnp.bfloat16),
            pltpu.VMEM(comm_shape, jnp.bfloat16),
            pltpu.VMEM(comm_shape, jnp.bfloat16),
            pltpu.VMEM(comm_shape, jnp.bfloat16),
            pltpu.VMEM(comm_shape, jnp.bfloat16),
            pltpu.SemaphoreType.DMA(sem_shape),
            pltpu.SemaphoreType.DMA(sem_shape),
            pltpu.SemaphoreType.DMA(sem_shape),
            pltpu.SemaphoreType.DMA(sem_shape),
        ],
        compiler_params=pltpu.CompilerParams(
            collective_id=0,
            vmem_limit_bytes=100 * 1024 * 1024,
        ),
    )(x, dy)


# device time: 53941 ns/iter; 1.8697x vs baseline; 1.0125x over previous
import jax
import jax.numpy as jnp
from jax import lax
from jax.experimental import pallas as pl
from jax.experimental.pallas import tpu as pltpu

N_DEV = 4
NB = 2


def kernel(x, dy):
    k_per, m = x.shape
    _, n = dy.shape
    chunk = m // N_DEV
    half = n // 2
    bw = half // NB

    def body(x_ref, dy_ref, out_ref, dyb_ref,
             send_r, recv_r, send_l, recv_l,
             ssem_r, rsem_r, ssem_l, rsem_l):
        p = lax.axis_index("i")
        left = lax.rem(p + N_DEV - 1, N_DEV)
        right = lax.rem(p + 1, N_DEV)

        barrier_sem = pltpu.get_barrier_semaphore()
        for nbr in (left, right):
            pl.semaphore_signal(
                barrier_sem, inc=1,
                device_id=(nbr,), device_id_type=pl.DeviceIdType.MESH,
            )
        pl.semaphore_wait(barrier_sem, 2)

        def part_block(c, col0, convert_dy=False):
            xa = x_ref[:, pl.ds(c * chunk, chunk)].astype(jnp.bfloat16)
            if convert_dy:
                db = dy_ref[:, col0:col0 + bw].astype(jnp.bfloat16)
                dyb_ref[:, col0:col0 + bw] = db
            else:
                db = dyb_ref[:, col0:col0 + bw]
            return lax.dot_general(
                xa, db, (((0,), (0,)), ((), ())),
                preferred_element_type=jnp.float32,
            )

        def make_rdma(h, b, direction):
            if direction == "r":
                return pltpu.make_async_remote_copy(
                    src_ref=send_r.at[h, b], dst_ref=recv_r.at[h, b],
                    send_sem=ssem_r.at[h, b], recv_sem=rsem_r.at[h, b],
                    device_id=(right,),
                    device_id_type=pl.DeviceIdType.MESH,
                )
            return pltpu.make_async_remote_copy(
                src_ref=send_l.at[h, b], dst_ref=recv_l.at[h, b],
                send_sem=ssem_l.at[h, b], recv_sem=rsem_l.at[h, b],
                device_id=(left,),
                device_id_type=pl.DeviceIdType.MESH,
            )

        c_send_r0 = lax.rem(p + 3, N_DEV)
        c_send_l0 = lax.rem(p + 1, N_DEV)

        for b in range(NB):
            send_r[0, b, :, :] = (
                part_block(c_send_r0, b * bw, convert_dy=True)
                .astype(jnp.bfloat16))
            make_rdma(0, b, "r").start()
            send_l[0, b, :, :] = (
                part_block(c_send_l0, half + b * bw, convert_dy=True)
                .astype(jnp.bfloat16))
            make_rdma(0, b, "l").start()

        for h in range(N_DEV - 1):
            c_r = lax.rem(p + 2 - h + N_DEV, N_DEV)
            c_l = lax.rem(p + 2 + h, N_DEV)
            for b in range(NB):
                part_r = part_block(c_r, b * bw)
                part_l = part_block(c_l, half + b * bw)

                rdma_r = make_rdma(h, b, "r")
                rdma_l = make_rdma(h, b, "l")
                rdma_r.wait()
                tot_r = recv_r[h, b, :, :].astype(jnp.float32) + part_r
                if h < N_DEV - 2:
                    send_r[h + 1, b, :, :] = tot_r.astype(jnp.bfloat16)
                    make_rdma(h + 1, b, "r").start()
                else:
                    out_ref[:, b * bw:(b + 1) * bw] = tot_r

                rdma_l.wait()
                tot_l = recv_l[h, b, :, :].astype(jnp.float32) + part_l
                if h < N_DEV - 2:
                    send_l[h + 1, b, :, :] = tot_l.astype(jnp.bfloat16)
                    make_rdma(h + 1, b, "l").start()
                else:
                    out_ref[:, half + b * bw:half + (b + 1) * bw] = tot_l

    comm_shape = (N_DEV - 1, NB, chunk, bw)
    sem_shape = (N_DEV - 1, NB)
    return pl.pallas_call(
        body,
        out_shape=jax.ShapeDtypeStruct((chunk, n), jnp.float32),
        in_specs=[
            pl.BlockSpec(memory_space=pltpu.VMEM),
            pl.BlockSpec(memory_space=pltpu.VMEM),
        ],
        out_specs=pl.BlockSpec(memory_space=pltpu.VMEM),
        scratch_shapes=[
            pltpu.VMEM((k_per, n), jnp.bfloat16),
            pltpu.VMEM(comm_shape, jnp.bfloat16),
            pltpu.VMEM(comm_shape, jnp.bfloat16),
            pltpu.VMEM(comm_shape, jnp.bfloat16),
            pltpu.VMEM(comm_shape, jnp.bfloat16),
            pltpu.SemaphoreType.DMA(sem_shape),
            pltpu.SemaphoreType.DMA(sem_shape),
            pltpu.SemaphoreType.DMA(sem_shape),
            pltpu.SemaphoreType.DMA(sem_shape),
        ],
        compiler_params=pltpu.CompilerParams(
            collective_id=0,
            vmem_limit_bytes=100 * 1024 * 1024,
        ),
    )(x, dy)


# device time: 53567 ns/iter; 1.8827x vs baseline; 1.0070x over previous
import os

import jax
import jax.numpy as jnp
from jax import lax
from jax.experimental import pallas as pl
from jax.experimental.pallas import tpu as pltpu

N_DEV = 4
NB = 4

_KDBG = os.environ.get("KDBG", "")


def kernel(x, dy):
    k_per, m = x.shape
    _, n = dy.shape
    chunk = m // N_DEV
    half = n // 2
    bw = half // NB

    def body(x_ref, dy_ref, out_ref, dyb_ref,
             send_r, recv_r, send_l, recv_l,
             ssem_r, rsem_r, ssem_l, rsem_l):
        p = lax.axis_index("i")
        left = lax.rem(p + N_DEV - 1, N_DEV)
        right = lax.rem(p + 1, N_DEV)

        if _KDBG != "nocomm":
            barrier_sem = pltpu.get_barrier_semaphore()
            for nbr in (left, right):
                pl.semaphore_signal(
                    barrier_sem, inc=1,
                    device_id=(nbr,), device_id_type=pl.DeviceIdType.MESH,
                )
            pl.semaphore_wait(barrier_sem, 2)

        if _KDBG == "barrier":
            out_ref[:, :] = jnp.zeros((chunk, n), jnp.float32)
            return

        def part_block(c, col0, convert_dy=False):
            if _KDBG == "nocompute":
                return jnp.zeros((chunk, bw), jnp.float32)
            xa = x_ref[:, pl.ds(c * chunk, chunk)].astype(jnp.bfloat16)
            if convert_dy:
                db = dy_ref[:, col0:col0 + bw].astype(jnp.bfloat16)
                dyb_ref[:, col0:col0 + bw] = db
            else:
                db = dyb_ref[:, col0:col0 + bw]
            return lax.dot_general(
                xa, db, (((0,), (0,)), ((), ())),
                preferred_element_type=jnp.float32,
            )

        def make_rdma(h, b, direction):
            if _KDBG.startswith(("forward", "fwduni")) and h > 0:
                if direction == "r":
                    return pltpu.make_async_remote_copy(
                        src_ref=recv_r.at[h - 1, b], dst_ref=recv_r.at[h, b],
                        send_sem=ssem_r.at[h, b], recv_sem=rsem_r.at[h, b],
                        device_id=(right,),
                        device_id_type=pl.DeviceIdType.MESH,
                    )
                return pltpu.make_async_remote_copy(
                    src_ref=recv_l.at[h - 1, b], dst_ref=recv_l.at[h, b],
                    send_sem=ssem_l.at[h, b], recv_sem=rsem_l.at[h, b],
                    device_id=(left,),
                    device_id_type=pl.DeviceIdType.MESH,
                )
            if _KDBG == "nocomm":
                class _Dummy:
                    def start(self):
                        pass

                    def wait(self):
                        pass

                return _Dummy()
            if direction == "r":
                return pltpu.make_async_remote_copy(
                    src_ref=send_r.at[h, b], dst_ref=recv_r.at[h, b],
                    send_sem=ssem_r.at[h, b], recv_sem=rsem_r.at[h, b],
                    device_id=(right,),
                    device_id_type=pl.DeviceIdType.MESH,
                )
            return pltpu.make_async_remote_copy(
                src_ref=send_l.at[h, b], dst_ref=recv_l.at[h, b],
                send_sem=ssem_l.at[h, b], recv_sem=rsem_l.at[h, b],
                device_id=(left,),
                device_id_type=pl.DeviceIdType.MESH,
            )

        c_send_r0 = lax.rem(p + 3, N_DEV)
        c_send_l0 = lax.rem(p + 1, N_DEV)

        _blocks = range(NB // 2) if _KDBG == "fwduni2" else range(NB)
        for b in _blocks:
            if not _KDBG.startswith(("forward", "fwduni")):
                send_r[0, b, :, :] = (
                    part_block(c_send_r0, b * bw, convert_dy=True)
                    .astype(jnp.bfloat16))
            make_rdma(0, b, "r").start()
            if not _KDBG.startswith(("forward", "fwduni")):
                send_l[0, b, :, :] = (
                    part_block(c_send_l0, half + b * bw, convert_dy=True)
                    .astype(jnp.bfloat16))
            if not _KDBG.startswith("fwduni"):
                make_rdma(0, b, "l").start()

        for h in range(N_DEV - 1):
            c_r = lax.rem(p + 2 - h + N_DEV, N_DEV)
            c_l = lax.rem(p + 2 + h, N_DEV)
            for b in _blocks:
                if _KDBG.startswith(("forward", "fwduni")):
                    rdma_r = make_rdma(h, b, "r")
                    rdma_r.wait()
                    if _KDBG == "forward":
                        make_rdma(h, b, "l").wait()
                    if h < N_DEV - 2:
                        make_rdma(h + 1, b, "r").start()
                        if _KDBG == "forward":
                            make_rdma(h + 1, b, "l").start()
                    else:
                        out_ref[:, b * bw:(b + 1) * bw] = (
                            recv_r[h, b, :, :].astype(jnp.float32))
                        out_ref[:, half + b * bw:half + (b + 1) * bw] = (
                            recv_l[h, b, :, :].astype(jnp.float32))
                    continue
                part_r = part_block(c_r, b * bw)
                part_l = part_block(c_l, half + b * bw)

                rdma_r = make_rdma(h, b, "r")
                rdma_l = make_rdma(h, b, "l")
                rdma_r.wait()
                tot_r = recv_r[h, b, :, :].astype(jnp.float32) + part_r
                if h < N_DEV - 2:
                    send_r[h + 1, b, :, :] = tot_r.astype(jnp.bfloat16)
                    make_rdma(h + 1, b, "r").start()
                else:
                    out_ref[:, b * bw:(b + 1) * bw] = tot_r

                rdma_l.wait()
                tot_l = recv_l[h, b, :, :].astype(jnp.float32) + part_l
                if h < N_DEV - 2:
                    send_l[h + 1, b, :, :] = tot_l.astype(jnp.bfloat16)
                    make_rdma(h + 1, b, "l").start()
                else:
                    out_ref[:, half + b * bw:half + (b + 1) * bw] = tot_l

    comm_shape = (N_DEV - 1, NB, chunk, bw)
    sem_shape = (N_DEV - 1, NB)
    return pl.pallas_call(
        body,
        out_shape=jax.ShapeDtypeStruct((chunk, n), jnp.float32),
        in_specs=[
            pl.BlockSpec(memory_space=pltpu.VMEM),
            pl.BlockSpec(memory_space=pltpu.VMEM),
        ],
        out_specs=pl.BlockSpec(memory_space=pltpu.VMEM),
        scratch_shapes=[
            pltpu.VMEM((k_per, n), jnp.bfloat16),
            pltpu.VMEM(comm_shape, jnp.bfloat16),
            pltpu.VMEM(comm_shape, jnp.bfloat16),
            pltpu.VMEM(comm_shape, jnp.bfloat16),
            pltpu.VMEM(comm_shape, jnp.bfloat16),
            pltpu.SemaphoreType.DMA(sem_shape),
            pltpu.SemaphoreType.DMA(sem_shape),
            pltpu.SemaphoreType.DMA(sem_shape),
            pltpu.SemaphoreType.DMA(sem_shape),
        ],
        compiler_params=pltpu.CompilerParams(
            collective_id=0,
            vmem_limit_bytes=100 * 1024 * 1024,
        ),
    )(x, dy)
